# Initial kernel scaffold; baseline (speedup 1.0000x reference)
import jax
import jax.numpy as jnp
from jax.experimental import pallas as pl


def kernel(x, batch):
    raise NotImplementedError("write your pallas kernel here")


# speedup vs baseline: 3.2647x; 3.2647x over previous
import functools

import jax
import jax.numpy as jnp
from jax import lax
from jax.experimental import pallas as pl
from jax.experimental.pallas import tpu as pltpu
from jax.experimental.pallas import tpu_sc as plsc

N_ROWS = 100000
D = 128
S = 512
NC, NS, L = 2, 16, 16
NW = NC * NS
CHUNK = 200
NCHUNKS = N_ROWS // CHUNK
MAXC = -(-NCHUNKS // NW)
KPER = D // L


def _sc_partial(xf, ids):
  mesh = plsc.VectorSubcoreMesh(core_axis_name="c", subcore_axis_name="s")

  @functools.partial(
      pl.kernel,
      out_type=[
          jax.ShapeDtypeStruct((NW, S * D), jnp.float32),
          jax.ShapeDtypeStruct((NW, S), jnp.float32),
      ],
      mesh=mesh,
      compiler_params=pltpu.CompilerParams(needs_layout_passes=False),
      scratch_types=[
          pltpu.VMEM((CHUNK * D,), jnp.float32),
          pltpu.VMEM((CHUNK * D,), jnp.float32),
          pltpu.VMEM((CHUNK,), jnp.int32),
          pltpu.VMEM((CHUNK,), jnp.int32),
          pltpu.VMEM((S * D,), jnp.float32),
          pltpu.VMEM((S,), jnp.float32),
          pltpu.SemaphoreType.DMA,
          pltpu.SemaphoreType.DMA,
          pltpu.SemaphoreType.DMA,
          pltpu.SemaphoreType.DMA,
      ],
  )
  def k(x_hbm, ids_hbm, psum_hbm, pcnt_hbm, xv0, xv1, iv0, iv1, acc, cnt,
        sx0, sx1, si0, si1):
    wid = lax.axis_index("s") * NC + lax.axis_index("c")
    xv = (xv0, xv1)
    iv = (iv0, iv1)
    sx = (sx0, sx1)
    si = (si0, si1)

    def start(i):
      slot = i % 2
      ci = jnp.minimum(wid + NW * i, NCHUNKS - 1)
      row0 = ci * CHUNK
      cx = pltpu.async_copy(x_hbm.at[pl.ds(row0 * D, CHUNK * D)], xv[slot],
                            sx[slot])
      cid = pltpu.async_copy(ids_hbm.at[pl.ds(row0, CHUNK)], iv[slot],
                             si[slot])
      return cx, cid

    copies = [None, None]
    copies[0] = start(0)

    zf = jnp.zeros((L,), jnp.float32)

    def zbody(j, _):
      for u in range(16):
        acc[pl.ds((j * 16 + u) * L, L)] = zf
      return 0

    lax.fori_loop(0, S * D // (16 * L), zbody, 0)

    def zcnt(j, _):
      cnt[pl.ds(j * L, L)] = zf
      return 0

    lax.fori_loop(0, S // L, zcnt, 0)

    cols = [jnp.arange(kk * L, (kk + 1) * L, dtype=jnp.int32)
            for kk in range(KPER)]
    ones = jnp.ones((L,), jnp.float32)
    lane0 = jnp.arange(L, dtype=jnp.int32) == 0

    for i in range(MAXC):
      if i + 1 < MAXC:
        copies[(i + 1) % 2] = start(i + 1)
      cx, cid = copies[i % 2]
      cx.wait()
      cid.wait()
      slot = i % 2
      valid = wid + NW * i < NCHUNKS
      xvs, ivs = xv[slot], iv[slot]

      def row_body(r, _, xvs=xvs, ivs=ivs):
        idv = plsc.load_gather(ivs, [jnp.broadcast_to(r, (L,))])
        base = idv * D
        for kk in range(KPER):
          xk = xvs[pl.ds(r * D + kk * L, L)]
          plsc.addupdate_scatter(acc, [base + cols[kk]], xk)
        plsc.addupdate_scatter(cnt, [idv], ones, mask=lane0)
        return 0

      @pl.when(valid)
      def _():
        lax.fori_loop(0, CHUNK, row_body, 0)

    pltpu.sync_copy(acc, psum_hbm.at[wid])
    pltpu.sync_copy(cnt, pcnt_hbm.at[wid])

  return k(xf, ids)


def _tc_reduce(psum, pcnt):
  BS = 128

  def body(ps_ref, pc_ref, o_ref):
    s = jnp.sum(ps_ref[...], axis=0)
    c = jnp.sum(pc_ref[...], axis=0)
    m = s / jnp.clip(c, 1.0, None)[:, None]
    o_ref[...] = jnp.concatenate([s, m], axis=-1)

  return pl.pallas_call(
      body,
      grid=(S // BS,),
      in_specs=[
          pl.BlockSpec((NW, BS, D), lambda i: (0, i, 0)),
          pl.BlockSpec((NW, BS), lambda i: (0, i)),
      ],
      out_specs=pl.BlockSpec((BS, 2 * D), lambda i: (i, 0)),
      out_shape=jax.ShapeDtypeStruct((S, 2 * D), jnp.float32),
  )(psum, pcnt)


def kernel(x, batch):
  ids = batch.astype(jnp.int32)
  xf = x.reshape(N_ROWS * D)
  psum, pcnt = _sc_partial(xf, ids)
  return _tc_reduce(psum.reshape(NW, S, D), pcnt)


# speedup vs baseline: 3.2875x; 1.0070x over previous
import functools

import jax
import jax.numpy as jnp
from jax import lax
from jax.experimental import pallas as pl
from jax.experimental.pallas import tpu as pltpu
from jax.experimental.pallas import tpu_sc as plsc

N_ROWS = 100000
D = 128
S = 512
NC, NS, L = 2, 16, 16
NW = NC * NS
CHUNK = 200
NCHUNKS = N_ROWS // CHUNK
MAXC = -(-NCHUNKS // NW)
KPER = D // L
RUNROLL = 8


def _sc_partial(xf, ids):
  mesh = plsc.VectorSubcoreMesh(core_axis_name="c", subcore_axis_name="s")

  @functools.partial(
      pl.kernel,
      out_type=[
          jax.ShapeDtypeStruct((NW, S * D), jnp.float32),
          jax.ShapeDtypeStruct((NW, S), jnp.float32),
      ],
      mesh=mesh,
      compiler_params=pltpu.CompilerParams(needs_layout_passes=False),
      scratch_types=[
          pltpu.VMEM((CHUNK * D,), jnp.float32),
          pltpu.VMEM((CHUNK * D,), jnp.float32),
          pltpu.VMEM((CHUNK,), jnp.int32),
          pltpu.VMEM((CHUNK,), jnp.int32),
          pltpu.VMEM((S * D,), jnp.float32),
          pltpu.VMEM((S,), jnp.float32),
          pltpu.SemaphoreType.DMA,
          pltpu.SemaphoreType.DMA,
          pltpu.SemaphoreType.DMA,
          pltpu.SemaphoreType.DMA,
      ],
  )
  def k(x_hbm, ids_hbm, psum_hbm, pcnt_hbm, xv0, xv1, iv0, iv1, acc, cnt,
        sx0, sx1, si0, si1):
    wid = lax.axis_index("s") * NC + lax.axis_index("c")
    xv = (xv0, xv1)
    iv = (iv0, iv1)
    sx = (sx0, sx1)
    si = (si0, si1)

    def start(i):
      slot = i % 2
      ci = jnp.minimum(wid + NW * i, NCHUNKS - 1)
      row0 = ci * CHUNK
      cx = pltpu.async_copy(x_hbm.at[pl.ds(row0 * D, CHUNK * D)], xv[slot],
                            sx[slot])
      cid = pltpu.async_copy(ids_hbm.at[pl.ds(row0, CHUNK)], iv[slot],
                             si[slot])
      return cx, cid

    copies = [None, None]
    copies[0] = start(0)

    zf = jnp.zeros((L,), jnp.float32)

    def zbody(j, _):
      for u in range(16):
        acc[pl.ds((j * 16 + u) * L, L)] = zf
      return 0

    lax.fori_loop(0, S * D // (16 * L), zbody, 0)

    def zcnt(j, _):
      cnt[pl.ds(j * L, L)] = zf
      return 0

    lax.fori_loop(0, S // L, zcnt, 0)

    cols = [jnp.arange(kk * L, (kk + 1) * L, dtype=jnp.int32)
            for kk in range(KPER)]
    ones = jnp.ones((L,), jnp.float32)
    lane0 = jnp.arange(L, dtype=jnp.int32) == 0

    for i in range(MAXC):
      if i + 1 < MAXC:
        copies[(i + 1) % 2] = start(i + 1)
      cx, cid = copies[i % 2]
      cx.wait()
      cid.wait()
      slot = i % 2
      valid = wid + NW * i < NCHUNKS
      xvs, ivs = xv[slot], iv[slot]

      def row_body(j, _, xvs=xvs, ivs=ivs):
        for u in range(RUNROLL):
          r = j * RUNROLL + u
          idv = plsc.load_gather(ivs, [jnp.broadcast_to(r, (L,))])
          base = idv * D
          for kk in range(KPER):
            xk = xvs[pl.ds(r * D + kk * L, L)]
            plsc.addupdate_scatter(acc, [base + cols[kk]], xk)
          plsc.addupdate_scatter(cnt, [idv], ones, mask=lane0)
        return 0

      @pl.when(valid)
      def _():
        lax.fori_loop(0, CHUNK // RUNROLL, row_body, 0)

    pltpu.sync_copy(acc, psum_hbm.at[wid])
    pltpu.sync_copy(cnt, pcnt_hbm.at[wid])

  return k(xf, ids)


def _tc_reduce(psum, pcnt):
  BS = 128

  def body(ps_ref, pc_ref, o_ref):
    s = jnp.sum(ps_ref[...], axis=0)
    c = jnp.sum(pc_ref[...], axis=0)
    m = s / jnp.clip(c, 1.0, None)[:, None]
    o_ref[...] = jnp.concatenate([s, m], axis=-1)

  return pl.pallas_call(
      body,
      grid=(S // BS,),
      in_specs=[
          pl.BlockSpec((NW, BS, D), lambda i: (0, i, 0)),
          pl.BlockSpec((NW, BS), lambda i: (0, i)),
      ],
      out_specs=pl.BlockSpec((BS, 2 * D), lambda i: (i, 0)),
      out_shape=jax.ShapeDtypeStruct((S, 2 * D), jnp.float32),
  )(psum, pcnt)


def kernel(x, batch):
  ids = batch.astype(jnp.int32)
  xf = x.reshape(N_ROWS * D)
  psum, pcnt = _sc_partial(xf, ids)
  return _tc_reduce(psum.reshape(NW, S, D), pcnt)


# speedup vs baseline: 5.8043x; 1.7656x over previous
import functools

import jax
import jax.numpy as jnp
from jax import lax
from jax.experimental import pallas as pl
from jax.experimental.pallas import tpu as pltpu
from jax.experimental.pallas import tpu_sc as plsc

N_ROWS = 100000
D = 128
S = 512
NC, NS, L = 2, 16, 16
NW = NC * NS
CHUNK = 200
NCHUNKS = N_ROWS // CHUNK
MAXC = -(-NCHUNKS // NW)
KPER = D // L
RUNROLL = 8


def _sc_partial(xf, ids):
  mesh = plsc.VectorSubcoreMesh(core_axis_name="c", subcore_axis_name="s")

  @functools.partial(
      pl.kernel,
      out_type=[
          jax.ShapeDtypeStruct((NW, S * D), jnp.float32),
          jax.ShapeDtypeStruct((NW, S), jnp.float32),
      ],
      mesh=mesh,
      compiler_params=pltpu.CompilerParams(needs_layout_passes=False),
      scratch_types=[
          pltpu.VMEM((CHUNK * D,), jnp.float32),
          pltpu.VMEM((CHUNK * D,), jnp.float32),
          pltpu.VMEM((CHUNK,), jnp.int32),
          pltpu.VMEM((CHUNK,), jnp.int32),
          pltpu.VMEM((S * D,), jnp.float32),
          pltpu.VMEM((S,), jnp.float32),
          pltpu.SemaphoreType.DMA,
          pltpu.SemaphoreType.DMA,
          pltpu.SemaphoreType.DMA,
          pltpu.SemaphoreType.DMA,
      ],
  )
  def k(x_hbm, ids_hbm, psum_hbm, pcnt_hbm, xv0, xv1, iv0, iv1, acc, cnt,
        sx0, sx1, si0, si1):
    wid = lax.axis_index("s") * NC + lax.axis_index("c")
    xv = (xv0, xv1)
    iv = (iv0, iv1)
    sx = (sx0, sx1)
    si = (si0, si1)

    def start(i):
      slot = i % 2
      ci = jnp.minimum(wid + NW * i, NCHUNKS - 1)
      row0 = ci * CHUNK
      cx = pltpu.async_copy(x_hbm.at[pl.ds(row0 * D, CHUNK * D)], xv[slot],
                            sx[slot])
      cid = pltpu.async_copy(ids_hbm.at[pl.ds(row0, CHUNK)], iv[slot],
                             si[slot])
      return cx, cid

    copies = [None, None]
    copies[0] = start(0)

    zf = jnp.zeros((L,), jnp.float32)

    def zbody(j, _):
      for u in range(16):
        acc[pl.ds((j * 16 + u) * L, L)] = zf
      return 0

    lax.fori_loop(0, S * D // (16 * L), zbody, 0)

    def zcnt(j, _):
      cnt[pl.ds(j * L, L)] = zf
      return 0

    lax.fori_loop(0, S // L, zcnt, 0)

    cols = [jnp.arange(kk * L, (kk + 1) * L, dtype=jnp.int32)
            for kk in range(KPER)]
    ones = jnp.ones((L,), jnp.float32)
    lane0 = jnp.arange(L, dtype=jnp.int32) == 0
    lanes = [jnp.full((L,), u, dtype=jnp.int32) for u in range(L)]

    def do_row(xvs, r, idu):
      base = idu * D
      xs = [xvs[pl.ds(r * D + kk * L, L)] for kk in range(KPER)]
      for kk in range(KPER):
        plsc.addupdate_scatter(acc, [base + cols[kk]], xs[kk])
      plsc.addupdate_scatter(cnt, [idu], ones, mask=lane0)

    NGRP = CHUNK // L
    TAIL = CHUNK - NGRP * L

    for i in range(MAXC):
      if i + 1 < MAXC:
        copies[(i + 1) % 2] = start(i + 1)
      cx, cid = copies[i % 2]
      cx.wait()
      cid.wait()
      slot = i % 2
      valid = wid + NW * i < NCHUNKS
      xvs, ivs = xv[slot], iv[slot]

      def grp_body(j, _, xvs=xvs, ivs=ivs):
        idvec = ivs[pl.ds(j * L, L)]
        for u in range(L):
          idu = jnp.take_along_axis(idvec, lanes[u], 0)
          do_row(xvs, j * L + u, idu)
        return 0

      @pl.when(valid)
      def _():
        lax.fori_loop(0, NGRP, grp_body, 0)
        idvec = ivs[pl.ds(CHUNK - L, L)]
        for u in range(L - TAIL, L):
          idu = jnp.take_along_axis(idvec, lanes[u], 0)
          do_row(xvs, CHUNK - L + u, idu)

    pltpu.sync_copy(acc, psum_hbm.at[wid])
    pltpu.sync_copy(cnt, pcnt_hbm.at[wid])

  return k(xf, ids)


def _tc_reduce(psum, pcnt):
  BS = 128

  def body(ps_ref, pc_ref, o_ref):
    s = jnp.sum(ps_ref[...], axis=0)
    c = jnp.sum(pc_ref[...], axis=0)
    m = s / jnp.clip(c, 1.0, None)[:, None]
    o_ref[...] = jnp.concatenate([s, m], axis=-1)

  return pl.pallas_call(
      body,
      grid=(S // BS,),
      in_specs=[
          pl.BlockSpec((NW, BS, D), lambda i: (0, i, 0)),
          pl.BlockSpec((NW, BS), lambda i: (0, i)),
      ],
      out_specs=pl.BlockSpec((BS, 2 * D), lambda i: (i, 0)),
      out_shape=jax.ShapeDtypeStruct((S, 2 * D), jnp.float32),
  )(psum, pcnt)


def kernel(x, batch):
  ids = batch.astype(jnp.int32)
  xf = x.reshape(N_ROWS * D)
  psum, pcnt = _sc_partial(xf, ids)
  return _tc_reduce(psum.reshape(NW, S, D), pcnt)


# speedup vs baseline: 6.4643x; 1.1137x over previous
import functools

import jax
import jax.numpy as jnp
from jax import lax
from jax.experimental import pallas as pl
from jax.experimental.pallas import tpu as pltpu
from jax.experimental.pallas import tpu_sc as plsc

N_ROWS = 100000
D = 128
S = 512
NC, NS, L = 2, 16, 16
NW = NC * NS
CHUNK = 200
NCHUNKS = N_ROWS // CHUNK
MAXC = -(-NCHUNKS // NW)
KPER = D // L
RUNROLL = 8


def _sc_partial(xf, ids):
  mesh = plsc.VectorSubcoreMesh(core_axis_name="c", subcore_axis_name="s")

  @functools.partial(
      pl.kernel,
      out_type=[
          jax.ShapeDtypeStruct((NW, S * D), jnp.float32),
          jax.ShapeDtypeStruct((NW, S), jnp.float32),
      ],
      mesh=mesh,
      compiler_params=pltpu.CompilerParams(needs_layout_passes=False),
      scratch_types=[
          pltpu.VMEM((CHUNK * D,), jnp.float32),
          pltpu.VMEM((CHUNK * D,), jnp.float32),
          pltpu.VMEM((CHUNK,), jnp.int32),
          pltpu.VMEM((CHUNK,), jnp.int32),
          pltpu.VMEM((S * D,), jnp.float32),
          pltpu.VMEM((S,), jnp.float32),
          pltpu.SemaphoreType.DMA,
          pltpu.SemaphoreType.DMA,
          pltpu.SemaphoreType.DMA,
          pltpu.SemaphoreType.DMA,
      ],
  )
  def k(x_hbm, ids_hbm, psum_hbm, pcnt_hbm, xv0, xv1, iv0, iv1, acc, cnt,
        sx0, sx1, si0, si1):
    wid = lax.axis_index("s") * NC + lax.axis_index("c")
    xv = (xv0, xv1)
    iv = (iv0, iv1)
    sx = (sx0, sx1)
    si = (si0, si1)

    def start(slot, i):
      valid = wid + NW * i < NCHUNKS

      @pl.when(valid)
      def _():
        row0 = (wid + NW * i) * CHUNK
        pltpu.async_copy(x_hbm.at[pl.ds(row0 * D, CHUNK * D)], xv[slot],
                         sx[slot])
        pltpu.async_copy(ids_hbm.at[pl.ds(row0, CHUNK)], iv[slot], si[slot])

    def wait(slot, i):
      valid = wid + NW * i < NCHUNKS

      @pl.when(valid)
      def _():
        pltpu.make_async_copy(x_hbm.at[pl.ds(0, CHUNK * D)], xv[slot],
                              sx[slot]).wait()
        pltpu.make_async_copy(ids_hbm.at[pl.ds(0, CHUNK)], iv[slot],
                              si[slot]).wait()

    start(0, 0)

    zf = jnp.zeros((L,), jnp.float32)

    def zbody(j, _):
      for u in range(16):
        acc[pl.ds((j * 16 + u) * L, L)] = zf
      return 0

    lax.fori_loop(0, S * D // (16 * L), zbody, 0)

    def zcnt(j, _):
      cnt[pl.ds(j * L, L)] = zf
      return 0

    lax.fori_loop(0, S // L, zcnt, 0)

    cols = [jnp.arange(kk * L, (kk + 1) * L, dtype=jnp.int32)
            for kk in range(KPER)]
    ones = jnp.ones((L,), jnp.float32)
    lane0 = jnp.arange(L, dtype=jnp.int32) == 0
    lanes = [jnp.full((L,), u, dtype=jnp.int32) for u in range(L)]

    def load_row(xvs, r):
      return [xvs[pl.ds(r * D + kk * L, L)] for kk in range(KPER)]

    def scatter_row(idu, xs):
      base = idu * D
      for kk in range(KPER):
        plsc.addupdate_scatter(acc, [base + cols[kk]], xs[kk])
      plsc.addupdate_scatter(cnt, [idu], ones, mask=lane0)

    NGRP = CHUNK // L
    TAIL = CHUNK - NGRP * L

    def pipelined_rows(xvs, idvec, r0, us):
      xs = load_row(xvs, r0 + us[0])
      for n, u in enumerate(us):
        idu = jnp.take_along_axis(idvec, lanes[u], 0)
        nxt = load_row(xvs, r0 + us[n + 1]) if n + 1 < len(us) else None
        scatter_row(idu, xs)
        xs = nxt

    def process(slot, i):
      valid = wid + NW * i < NCHUNKS
      xvs, ivs = xv[slot], iv[slot]

      @pl.when(valid)
      def _():
        @plsc.parallel_loop(0, NGRP, 1)
        def _grp(j):
          idvec = ivs[pl.ds(j * L, L)]
          for u in range(L):
            idu = jnp.take_along_axis(idvec, lanes[u], 0)
            xs = load_row(xvs, j * L + u)
            scatter_row(idu, xs)

        idvec = ivs[pl.ds(CHUNK - L, L)]
        pipelined_rows(xvs, idvec, CHUNK - L, list(range(L - TAIL, L)))

    start(1, 1)

    def outer(t, _):
      i0 = 2 * t
      wait(0, i0)
      process(0, i0)
      start(0, i0 + 2)
      wait(1, i0 + 1)
      process(1, i0 + 1)
      start(1, i0 + 3)
      return 0

    lax.fori_loop(0, MAXC // 2, outer, 0)

    pltpu.sync_copy(acc, psum_hbm.at[wid])
    pltpu.sync_copy(cnt, pcnt_hbm.at[wid])

  return k(xf, ids)


def _tc_reduce(psum, pcnt):
  BS = 128

  def body(ps_ref, pc_ref, o_ref):
    s = jnp.sum(ps_ref[...], axis=0)
    c = jnp.sum(pc_ref[...], axis=0)
    m = s / jnp.clip(c, 1.0, None)[:, None]
    o_ref[...] = jnp.concatenate([s, m], axis=-1)

  return pl.pallas_call(
      body,
      grid=(S // BS,),
      in_specs=[
          pl.BlockSpec((NW, BS, D), lambda i: (0, i, 0)),
          pl.BlockSpec((NW, BS), lambda i: (0, i)),
      ],
      out_specs=pl.BlockSpec((BS, 2 * D), lambda i: (i, 0)),
      out_shape=jax.ShapeDtypeStruct((S, 2 * D), jnp.float32),
  )(psum, pcnt)


def kernel(x, batch):
  ids = batch.astype(jnp.int32)
  xf = x.reshape(N_ROWS * D)
  psum, pcnt = _sc_partial(xf, ids)
  return _tc_reduce(psum.reshape(NW, S, D), pcnt)
